# Initial kernel scaffold; baseline (speedup 1.0000x reference)
import jax
import jax.numpy as jnp
from jax.experimental import pallas as pl


def kernel(hidden_states, W):
    raise NotImplementedError("write your pallas kernel here")


# speedup vs baseline: 4.6610x; 4.6610x over previous
import functools

import jax
import jax.numpy as jnp
from jax import lax
from jax.experimental import pallas as pl
from jax.experimental.pallas import tpu as pltpu
from jax.experimental.pallas import tpu_sc as plsc

N_TOKENS = 4096
HIDDEN_DIM = 2048
NUM_EXPERTS = 8
CAPACITY_FACTOR = 1.25

NC = 2
NS = 16
L = 16
NW = NC * NS
TPW = N_TOKENS // NW
CPW = TPW // L
NCHUNKS = N_TOKENS // L

BT = 512
NBLK = N_TOKENS // BT


def _capacity(num_tokens: int, num_experts: int) -> int:
    min_capacity_per_expert = 4
    base_capacity = int(CAPACITY_FACTOR * num_tokens / num_experts)
    if base_capacity < min_capacity_per_expert:
        adjusted_cf = min_capacity_per_expert * num_experts / max(num_tokens, 1)
        return max(min_capacity_per_expert,
                   int(adjusted_cf * num_tokens / num_experts))
    return base_capacity


CAP = _capacity(N_TOKENS, NUM_EXPERTS)
ECAP = NUM_EXPERTS * CAP
SENTINEL = ECAP

COLB = 2560
NCOL = ECAP // COLB


def _router_body(x_ref, wt_ref, probs_ref, gate_ref, idx_ref, part_ref):
    x = x_ref[...]
    wt = wt_ref[...]
    logits = jnp.dot(x, wt, preferred_element_type=jnp.float32)
    m = jnp.max(logits, axis=-1, keepdims=True)
    e = jnp.exp(logits - m)
    probs = e / jnp.sum(e, axis=-1, keepdims=True)
    probs_ref[...] = probs
    gate = jnp.max(probs, axis=-1, keepdims=True)
    gate_ref[...] = gate
    eiota = lax.broadcasted_iota(jnp.int32, probs.shape, 1)
    idx = jnp.min(jnp.where(probs == gate, eiota, NUM_EXPERTS),
                  axis=-1, keepdims=True)
    idx_ref[...] = idx
    colsum = jnp.sum(probs, axis=0)
    zsum = jnp.sum(logits * logits)
    row = jnp.concatenate([colsum, jnp.full((8,), zsum, jnp.float32)], axis=0)
    part_ref[...] = row.reshape(1, 1, 16)


def _run_router(hidden_flat, wt):
    return pl.pallas_call(
        _router_body,
        grid=(NBLK,),
        in_specs=[
            pl.BlockSpec((BT, HIDDEN_DIM), lambda b: (b, 0)),
            pl.BlockSpec((HIDDEN_DIM, NUM_EXPERTS), lambda b: (0, 0)),
        ],
        out_specs=[
            pl.BlockSpec((BT, NUM_EXPERTS), lambda b: (b, 0)),
            pl.BlockSpec((BT, 1), lambda b: (b, 0)),
            pl.BlockSpec((BT, 1), lambda b: (b, 0)),
            pl.BlockSpec((1, 1, 16), lambda b: (b, 0, 0)),
        ],
        out_shape=[
            jax.ShapeDtypeStruct((N_TOKENS, NUM_EXPERTS), jnp.float32),
            jax.ShapeDtypeStruct((N_TOKENS, 1), jnp.float32),
            jax.ShapeDtypeStruct((N_TOKENS, 1), jnp.int32),
            jax.ShapeDtypeStruct((NBLK, 1, 16), jnp.float32),
        ],
    )(hidden_flat, wt)


def _route_sc_body(idx_hbm, gate_hbm, slot_hbm, counts_hbm, e_v, g_v, out_v,
                   cnt_v):
    wid = lax.axis_index("s") * NC + lax.axis_index("c")
    base = wid * TPW
    my_first = wid * CPW

    pltpu.sync_copy(idx_hbm, e_v)
    pltpu.sync_copy(gate_hbm, g_v)

    zero = jnp.int32(0)

    def count_body(c, carry):
        tot, pre = carry
        ev = e_v[pl.ds(c * L, L)]
        in_pre = jnp.where(c < my_first, jnp.int32(1), zero)
        new_tot = []
        new_pre = []
        for k in range(NUM_EXPERTS):
            s = jnp.sum((ev == k).astype(jnp.int32))
            new_tot.append(tot[k] + s)
            new_pre.append(pre[k] + s * in_pre)
        return tuple(new_tot), tuple(new_pre)

    init = (tuple(zero for _ in range(NUM_EXPERTS)),
            tuple(zero for _ in range(NUM_EXPERTS)))
    tot, pre = lax.fori_loop(0, NCHUNKS, count_body, init)

    ov = [tot[k] > CAP for k in range(NUM_EXPERTS)]
    any_ov = functools.reduce(jnp.logical_or, ov)

    lane = lax.broadcasted_iota(jnp.int32, (L,), 0)

    evs = [e_v[pl.ds((my_first + v) * L, L)] for v in range(CPW)]
    gvs = [g_v[pl.ds((my_first + v) * L, L)] for v in range(CPW)]

    run = list(pre)
    pos_vecs = []
    for v in range(CPW):
        ev = evs[v]
        pos = jnp.zeros((L,), jnp.int32)
        for k in range(NUM_EXPERTS):
            m = ev == k
            mi = m.astype(jnp.int32)
            cs = plsc.cumsum(mi)
            pos = jnp.where(m, run[k] + cs - 1, pos)
            run[k] = run[k] + jnp.sum(mi)
        pos_vecs.append(pos)

    def grank_true(_):
        def pair_body(c, accs):
            eu_vec = e_v[pl.ds(c * L, L)]
            gu_vec = g_v[pl.ds(c * L, L)]
            accs = list(accs)
            for j in range(L):
                eu = eu_vec[j]
                gu = gu_vec[j]
                u = c * L + j
                for v in range(CPW):
                    tvec = base + v * L + lane
                    same = evs[v] == eu
                    beat = (gu > gvs[v]) | ((gu == gvs[v]) & (u < tvec))
                    accs[v] = accs[v] + (same & beat).astype(jnp.int32)
            return tuple(accs)

        return lax.fori_loop(
            0, NCHUNKS, pair_body,
            tuple(jnp.zeros((L,), jnp.int32) for _ in range(CPW)))

    def grank_false(_):
        return tuple(jnp.zeros((L,), jnp.int32) for _ in range(CPW))

    granks = lax.cond(any_ov, grank_true, grank_false, None)

    for v in range(CPW):
        ev = evs[v]
        ovv = jnp.zeros((L,), jnp.int32)
        for k in range(NUM_EXPERTS):
            ovv = jnp.where(ev == k, jnp.where(ov[k], jnp.int32(1), zero), ovv)
        ovb = ovv == 1
        slot = jnp.where(ovb, granks[v], pos_vecs[v])
        keep = jnp.where(ovb, granks[v] < CAP, True)
        flat = jnp.where(keep, ev * CAP + slot, jnp.int32(SENTINEL))
        out_v[pl.ds(v * L, L)] = flat

    pltpu.sync_copy(out_v, slot_hbm.at[pl.ds(base, TPW)])

    @pl.when(wid == 0)
    def _():
        cvec = jnp.zeros((L,), jnp.int32)
        for k in range(NUM_EXPERTS):
            cvec = jnp.where(lane == k, tot[k], cvec)
        cnt_v[...] = cvec
        pltpu.sync_copy(cnt_v, counts_hbm)


def _run_route_sc(expert_idx, gate):
    mesh = plsc.VectorSubcoreMesh(core_axis_name="c", subcore_axis_name="s")
    return pl.kernel(
        _route_sc_body,
        out_type=[
            jax.ShapeDtypeStruct((N_TOKENS,), jnp.int32),
            jax.ShapeDtypeStruct((L,), jnp.int32),
        ],
        mesh=mesh,
        compiler_params=pltpu.CompilerParams(needs_layout_passes=False),
        scratch_types=[
            pltpu.VMEM((N_TOKENS,), jnp.int32),
            pltpu.VMEM((N_TOKENS,), jnp.float32),
            pltpu.VMEM((TPW,), jnp.int32),
            pltpu.VMEM((L,), jnp.int32),
        ],
    )(expert_idx, gate)


def _expand_body(slot_ref, gate_ref, disp_ref, comb_ref):
    slot = slot_ref[...]
    gate = gate_ref[...]
    ci = (lax.broadcasted_iota(jnp.int32, (BT, COLB), 1)
          + pl.program_id(1) * COLB)
    m = ci == slot
    disp_ref[...] = m.astype(jnp.float32)
    comb_ref[...] = jnp.where(m, gate, 0.0)


def _run_expand(slot2, gate2):
    return pl.pallas_call(
        _expand_body,
        grid=(NBLK, NCOL),
        in_specs=[
            pl.BlockSpec((BT, 1), lambda b, j: (b, 0)),
            pl.BlockSpec((BT, 1), lambda b, j: (b, 0)),
        ],
        out_specs=[
            pl.BlockSpec((BT, COLB), lambda b, j: (b, j)),
            pl.BlockSpec((BT, COLB), lambda b, j: (b, j)),
        ],
        out_shape=[
            jax.ShapeDtypeStruct((N_TOKENS, ECAP), jnp.float32),
            jax.ShapeDtypeStruct((N_TOKENS, ECAP), jnp.float32),
        ],
    )(slot2, gate2)


def kernel(hidden_states, W):
    B, S, D = hidden_states.shape
    n = B * S
    E = W.shape[0]
    assert n == N_TOKENS and D == HIDDEN_DIM and E == NUM_EXPERTS

    hidden_flat = hidden_states.reshape(n, D)
    wt = W.T

    probs, gate2, idx2, partials = _run_router(hidden_flat, wt)

    slots, counts16 = _run_route_sc(idx2.reshape(n), gate2.reshape(n))
    counts = counts16[:E]

    disp_flat, comb_flat = _run_expand(slots.reshape(n, 1), gate2)
    dispatch = disp_flat.reshape(n, E, CAP)
    combine = comb_flat.reshape(n, E, CAP)

    usage = jnp.minimum(counts, CAP).astype(jnp.float32)
    psum = partials[:, 0, :E].sum(axis=0)
    zsum = partials[:, 0, E].sum()
    prob_per_expert = psum / n
    usage_per_expert = counts.astype(jnp.float32) / n
    load_loss = E * jnp.sum(prob_per_expert * usage_per_expert)
    z_loss = zsum / (n * E)
    dropped = jnp.maximum(0.0, n - usage.sum())

    return (dispatch, combine, jnp.array(CAP, dtype=jnp.int32), probs, usage,
            load_loss, z_loss, dropped)


# speedup vs baseline: 9.8765x; 2.1190x over previous
import functools

import jax
import jax.numpy as jnp
from jax import lax
from jax.experimental import pallas as pl
from jax.experimental.pallas import tpu as pltpu
from jax.experimental.pallas import tpu_sc as plsc

N_TOKENS = 4096
HIDDEN_DIM = 2048
NUM_EXPERTS = 8
CAPACITY_FACTOR = 1.25

NC = 2
NS = 16
L = 16
NW = NC * NS
TPW = N_TOKENS // NW
CPW = TPW // L
NCHUNKS = N_TOKENS // L

BT = 512
NBLK = N_TOKENS // BT


def _capacity(num_tokens: int, num_experts: int) -> int:
    min_capacity_per_expert = 4
    base_capacity = int(CAPACITY_FACTOR * num_tokens / num_experts)
    if base_capacity < min_capacity_per_expert:
        adjusted_cf = min_capacity_per_expert * num_experts / max(num_tokens, 1)
        return max(min_capacity_per_expert,
                   int(adjusted_cf * num_tokens / num_experts))
    return base_capacity


CAP = _capacity(N_TOKENS, NUM_EXPERTS)
ECAP = NUM_EXPERTS * CAP
SENTINEL = ECAP

COLB = 2560
NCOL = ECAP // COLB


def _router_body(x_ref, wt_ref, probs_ref, gate_ref, idx_ref, part_ref):
    x = x_ref[...]
    wt = wt_ref[...]
    logits = jnp.dot(x, wt, preferred_element_type=jnp.float32)
    m = jnp.max(logits, axis=-1, keepdims=True)
    e = jnp.exp(logits - m)
    probs = e / jnp.sum(e, axis=-1, keepdims=True)
    probs_ref[...] = probs
    gate = jnp.max(probs, axis=-1, keepdims=True)
    gate_ref[...] = gate
    eiota = lax.broadcasted_iota(jnp.int32, probs.shape, 1)
    idx = jnp.min(jnp.where(probs == gate, eiota, NUM_EXPERTS),
                  axis=-1, keepdims=True)
    idx_ref[...] = idx
    colsum = jnp.sum(probs, axis=0)
    zsum = jnp.sum(logits * logits)
    row = jnp.concatenate([colsum, jnp.full((8,), zsum, jnp.float32)], axis=0)
    part_ref[...] = row.reshape(1, 1, 16)


def _run_router(hidden_flat, wt):
    return pl.pallas_call(
        _router_body,
        grid=(NBLK,),
        in_specs=[
            pl.BlockSpec((BT, HIDDEN_DIM), lambda b: (b, 0)),
            pl.BlockSpec((HIDDEN_DIM, NUM_EXPERTS), lambda b: (0, 0)),
        ],
        out_specs=[
            pl.BlockSpec((BT, NUM_EXPERTS), lambda b: (b, 0)),
            pl.BlockSpec((BT, 1), lambda b: (b, 0)),
            pl.BlockSpec((BT, 1), lambda b: (b, 0)),
            pl.BlockSpec((1, 1, 16), lambda b: (b, 0, 0)),
        ],
        out_shape=[
            jax.ShapeDtypeStruct((N_TOKENS, NUM_EXPERTS), jnp.float32),
            jax.ShapeDtypeStruct((N_TOKENS, 1), jnp.float32),
            jax.ShapeDtypeStruct((N_TOKENS, 1), jnp.int32),
            jax.ShapeDtypeStruct((NBLK, 1, 16), jnp.float32),
        ],
    )(hidden_flat, wt)


def _route_sc_body(idx_hbm, gate_hbm, slot_hbm, counts_hbm, e_v, g_v, out_v,
                   cnt_v):
    wid = lax.axis_index("s") * NC + lax.axis_index("c")
    base = wid * TPW
    my_first = wid * CPW

    pltpu.sync_copy(idx_hbm, e_v)
    pltpu.sync_copy(gate_hbm, g_v)

    zero = jnp.int32(0)

    def count_body(c, carry):
        tot, pre = carry
        ev = e_v[pl.ds(c * L, L)]
        in_pre = jnp.where(c < my_first, jnp.int32(1), zero)
        new_tot = []
        new_pre = []
        for k in range(NUM_EXPERTS):
            s = jnp.sum((ev == k).astype(jnp.int32))
            new_tot.append(tot[k] + s)
            new_pre.append(pre[k] + s * in_pre)
        return tuple(new_tot), tuple(new_pre)

    init = (tuple(zero for _ in range(NUM_EXPERTS)),
            tuple(zero for _ in range(NUM_EXPERTS)))
    tot, pre = lax.fori_loop(0, NCHUNKS, count_body, init)

    ov = [tot[k] > CAP for k in range(NUM_EXPERTS)]
    any_ov = functools.reduce(jnp.logical_or, ov)

    lane = lax.broadcasted_iota(jnp.int32, (L,), 0)

    evs = [e_v[pl.ds((my_first + v) * L, L)] for v in range(CPW)]
    gvs = [g_v[pl.ds((my_first + v) * L, L)] for v in range(CPW)]

    run = list(pre)
    pos_vecs = []
    for v in range(CPW):
        ev = evs[v]
        pos = jnp.zeros((L,), jnp.int32)
        for k in range(NUM_EXPERTS):
            m = ev == k
            mi = m.astype(jnp.int32)
            cs = plsc.cumsum(mi)
            pos = jnp.where(m, run[k] + cs - 1, pos)
            run[k] = run[k] + jnp.sum(mi)
        pos_vecs.append(pos)

    def grank_true(_):
        def pair_body(c, accs):
            eu_vec = e_v[pl.ds(c * L, L)]
            gu_vec = g_v[pl.ds(c * L, L)]
            accs = list(accs)
            for j in range(L):
                eu = eu_vec[j]
                gu = gu_vec[j]
                u = c * L + j
                for v in range(CPW):
                    tvec = base + v * L + lane
                    same = evs[v] == eu
                    beat = (gu > gvs[v]) | ((gu == gvs[v]) & (u < tvec))
                    accs[v] = accs[v] + (same & beat).astype(jnp.int32)
            return tuple(accs)

        return lax.fori_loop(
            0, NCHUNKS, pair_body,
            tuple(jnp.zeros((L,), jnp.int32) for _ in range(CPW)))

    def grank_false(_):
        return tuple(jnp.zeros((L,), jnp.int32) for _ in range(CPW))

    granks = lax.cond(any_ov, grank_true, grank_false, None)

    for v in range(CPW):
        ev = evs[v]
        ovv = jnp.zeros((L,), jnp.int32)
        for k in range(NUM_EXPERTS):
            ovv = jnp.where(ev == k, jnp.where(ov[k], jnp.int32(1), zero), ovv)
        ovb = ovv == 1
        slot = jnp.where(ovb, granks[v], pos_vecs[v])
        keep = jnp.where(ovb, granks[v] < CAP, True)
        flat = jnp.where(keep, ev * CAP + slot, jnp.int32(SENTINEL))
        out_v[pl.ds(v * L, L)] = flat

    pltpu.sync_copy(out_v, slot_hbm.at[pl.ds(base, TPW)])

    @pl.when(wid == 0)
    def _():
        cvec = jnp.zeros((L,), jnp.int32)
        for k in range(NUM_EXPERTS):
            cvec = jnp.where(lane == k, tot[k], cvec)
        cnt_v[...] = cvec
        pltpu.sync_copy(cnt_v, counts_hbm)


def _run_route_sc(expert_idx, gate):
    mesh = plsc.VectorSubcoreMesh(core_axis_name="c", subcore_axis_name="s")
    return pl.kernel(
        _route_sc_body,
        out_type=[
            jax.ShapeDtypeStruct((N_TOKENS,), jnp.int32),
            jax.ShapeDtypeStruct((L,), jnp.int32),
        ],
        mesh=mesh,
        compiler_params=pltpu.CompilerParams(needs_layout_passes=False),
        scratch_types=[
            pltpu.VMEM((N_TOKENS,), jnp.int32),
            pltpu.VMEM((N_TOKENS,), jnp.float32),
            pltpu.VMEM((TPW,), jnp.int32),
            pltpu.VMEM((L,), jnp.int32),
        ],
    )(expert_idx, gate)


BT3 = 256
NBLK3 = N_TOKENS // BT3


def _expand_body(slot_ref, gate_ref, disp_ref, comb_ref):
    slot = slot_ref[...].reshape(BT3, 1, 1)
    gate = gate_ref[...].reshape(BT3, 1, 1)
    shp = (BT3, NUM_EXPERTS, CAP)
    ci = (lax.broadcasted_iota(jnp.int32, shp, 1) * CAP
          + lax.broadcasted_iota(jnp.int32, shp, 2))
    m = ci == slot
    disp_ref[...] = m.astype(jnp.float32)
    comb_ref[...] = jnp.where(m, gate, 0.0)


def _run_expand(slot2, gate2):
    return pl.pallas_call(
        _expand_body,
        grid=(NBLK3,),
        in_specs=[
            pl.BlockSpec((BT3, 1), lambda b: (b, 0)),
            pl.BlockSpec((BT3, 1), lambda b: (b, 0)),
        ],
        out_specs=[
            pl.BlockSpec((BT3, NUM_EXPERTS, CAP), lambda b: (b, 0, 0)),
            pl.BlockSpec((BT3, NUM_EXPERTS, CAP), lambda b: (b, 0, 0)),
        ],
        out_shape=[
            jax.ShapeDtypeStruct((N_TOKENS, NUM_EXPERTS, CAP), jnp.float32),
            jax.ShapeDtypeStruct((N_TOKENS, NUM_EXPERTS, CAP), jnp.float32),
        ],
    )(slot2, gate2)


def kernel(hidden_states, W):
    B, S, D = hidden_states.shape
    n = B * S
    E = W.shape[0]
    assert n == N_TOKENS and D == HIDDEN_DIM and E == NUM_EXPERTS

    hidden_flat = hidden_states.reshape(n, D)
    wt = W.T

    probs, gate2, idx2, partials = _run_router(hidden_flat, wt)

    slots, counts16 = _run_route_sc(idx2.reshape(n), gate2.reshape(n))
    counts = counts16[:E]

    dispatch, combine = _run_expand(slots.reshape(n, 1), gate2)

    usage = jnp.minimum(counts, CAP).astype(jnp.float32)
    psum = partials[:, 0, :E].sum(axis=0)
    zsum = partials[:, 0, E].sum()
    prob_per_expert = psum / n
    usage_per_expert = counts.astype(jnp.float32) / n
    load_loss = E * jnp.sum(prob_per_expert * usage_per_expert)
    z_loss = zsum / (n * E)
    dropped = jnp.maximum(0.0, n - usage.sum())

    return (dispatch, combine, jnp.array(CAP, dtype=jnp.int32), probs, usage,
            load_loss, z_loss, dropped)


# speedup vs baseline: 9.9861x; 1.0111x over previous
import functools

import jax
import jax.numpy as jnp
from jax import lax
from jax.experimental import pallas as pl
from jax.experimental.pallas import tpu as pltpu
from jax.experimental.pallas import tpu_sc as plsc

N_TOKENS = 4096
HIDDEN_DIM = 2048
NUM_EXPERTS = 8
CAPACITY_FACTOR = 1.25

NC = 2
NS = 16
L = 16
NW = NC * NS
TPW = N_TOKENS // NW
CPW = TPW // L
NCHUNKS = N_TOKENS // L

BT = 512
NBLK = N_TOKENS // BT


def _capacity(num_tokens: int, num_experts: int) -> int:
    min_capacity_per_expert = 4
    base_capacity = int(CAPACITY_FACTOR * num_tokens / num_experts)
    if base_capacity < min_capacity_per_expert:
        adjusted_cf = min_capacity_per_expert * num_experts / max(num_tokens, 1)
        return max(min_capacity_per_expert,
                   int(adjusted_cf * num_tokens / num_experts))
    return base_capacity


CAP = _capacity(N_TOKENS, NUM_EXPERTS)
ECAP = NUM_EXPERTS * CAP
SENTINEL = ECAP

COLB = 2560
NCOL = ECAP // COLB


def _router_body(x_ref, wt_ref, probs_ref, gate_ref, idx_ref, part_ref):
    x = x_ref[...]
    wt = wt_ref[...]
    logits = jnp.dot(x, wt, preferred_element_type=jnp.float32)
    m = jnp.max(logits, axis=-1, keepdims=True)
    e = jnp.exp(logits - m)
    probs = e / jnp.sum(e, axis=-1, keepdims=True)
    probs_ref[...] = probs
    gate = jnp.max(probs, axis=-1, keepdims=True)
    gate_ref[...] = gate
    eiota = lax.broadcasted_iota(jnp.int32, probs.shape, 1)
    idx = jnp.min(jnp.where(probs == gate, eiota, NUM_EXPERTS),
                  axis=-1, keepdims=True)
    idx_ref[...] = idx
    colsum = jnp.sum(probs, axis=0)
    zsum = jnp.sum(logits * logits)
    row = jnp.concatenate([colsum, jnp.full((8,), zsum, jnp.float32)], axis=0)
    part_ref[...] = row.reshape(1, 1, 16)


def _run_router(hidden_flat, wt):
    return pl.pallas_call(
        _router_body,
        grid=(NBLK,),
        in_specs=[
            pl.BlockSpec((BT, HIDDEN_DIM), lambda b: (b, 0)),
            pl.BlockSpec((HIDDEN_DIM, NUM_EXPERTS), lambda b: (0, 0)),
        ],
        out_specs=[
            pl.BlockSpec((BT, NUM_EXPERTS), lambda b: (b, 0)),
            pl.BlockSpec((BT, 1), lambda b: (b, 0)),
            pl.BlockSpec((BT, 1), lambda b: (b, 0)),
            pl.BlockSpec((1, 1, 16), lambda b: (b, 0, 0)),
        ],
        out_shape=[
            jax.ShapeDtypeStruct((N_TOKENS, NUM_EXPERTS), jnp.float32),
            jax.ShapeDtypeStruct((N_TOKENS, 1), jnp.float32),
            jax.ShapeDtypeStruct((N_TOKENS, 1), jnp.int32),
            jax.ShapeDtypeStruct((NBLK, 1, 16), jnp.float32),
        ],
    )(hidden_flat, wt)


def _route_sc_body(idx_hbm, gate_hbm, slot_hbm, counts_hbm, e_v, g_v, out_v,
                   cnt_v):
    wid = lax.axis_index("s") * NC + lax.axis_index("c")
    base = wid * TPW
    my_first = wid * CPW

    pltpu.sync_copy(idx_hbm, e_v)
    pltpu.sync_copy(gate_hbm, g_v)

    zero = jnp.int32(0)
    zv = jnp.zeros((L,), jnp.int32)

    def count_body(c, carry):
        tot, pre = carry
        ev = e_v[pl.ds(c * L, L)]
        in_pre = jnp.where(c < my_first, jnp.int32(1), zero)
        new_tot = []
        new_pre = []
        for k in range(NUM_EXPERTS):
            pc = plsc.all_reduce_population_count(ev == k)
            new_tot.append(tot[k] + pc)
            new_pre.append(pre[k] + pc * in_pre)
        return tuple(new_tot), tuple(new_pre)

    init = (tuple(zv for _ in range(NUM_EXPERTS)),
            tuple(zv for _ in range(NUM_EXPERTS)))
    tot_v, pre_v = lax.fori_loop(0, NCHUNKS, count_body, init)
    tot = [tot_v[k][0] for k in range(NUM_EXPERTS)]
    pre = [pre_v[k][0] for k in range(NUM_EXPERTS)]

    ov = [tot[k] > CAP for k in range(NUM_EXPERTS)]
    any_ov = functools.reduce(jnp.logical_or, ov)

    lane = lax.broadcasted_iota(jnp.int32, (L,), 0)

    evs = [e_v[pl.ds((my_first + v) * L, L)] for v in range(CPW)]
    gvs = [g_v[pl.ds((my_first + v) * L, L)] for v in range(CPW)]

    run = list(pre)
    pos_vecs = []
    for v in range(CPW):
        ev = evs[v]
        pos = jnp.zeros((L,), jnp.int32)
        for k in range(NUM_EXPERTS):
            m = ev == k
            mi = m.astype(jnp.int32)
            cs = plsc.cumsum(mi)
            pos = jnp.where(m, run[k] + cs - 1, pos)
            run[k] = run[k] + plsc.all_reduce_population_count(m)[0]
        pos_vecs.append(pos)

    def grank_true(_):
        def pair_body(c, accs):
            eu_vec = e_v[pl.ds(c * L, L)]
            gu_vec = g_v[pl.ds(c * L, L)]
            accs = list(accs)
            for j in range(L):
                eu = eu_vec[j]
                gu = gu_vec[j]
                u = c * L + j
                for v in range(CPW):
                    tvec = base + v * L + lane
                    same = evs[v] == eu
                    beat = (gu > gvs[v]) | ((gu == gvs[v]) & (u < tvec))
                    accs[v] = accs[v] + (same & beat).astype(jnp.int32)
            return tuple(accs)

        return lax.fori_loop(
            0, NCHUNKS, pair_body,
            tuple(jnp.zeros((L,), jnp.int32) for _ in range(CPW)))

    def grank_false(_):
        return tuple(jnp.zeros((L,), jnp.int32) for _ in range(CPW))

    granks = lax.cond(any_ov, grank_true, grank_false, None)

    for v in range(CPW):
        ev = evs[v]
        ovv = jnp.zeros((L,), jnp.int32)
        for k in range(NUM_EXPERTS):
            ovv = jnp.where(ev == k, jnp.where(ov[k], jnp.int32(1), zero), ovv)
        ovb = ovv == 1
        slot = jnp.where(ovb, granks[v], pos_vecs[v])
        keep = jnp.where(ovb, granks[v] < CAP, True)
        flat = jnp.where(keep, ev * CAP + slot, jnp.int32(SENTINEL))
        out_v[pl.ds(v * L, L)] = flat

    pltpu.sync_copy(out_v, slot_hbm.at[pl.ds(base, TPW)])

    @pl.when(wid == 0)
    def _():
        cvec = jnp.zeros((L,), jnp.int32)
        for k in range(NUM_EXPERTS):
            cvec = jnp.where(lane == k, tot[k], cvec)
        cnt_v[...] = cvec
        pltpu.sync_copy(cnt_v, counts_hbm)


def _run_route_sc(expert_idx, gate):
    mesh = plsc.VectorSubcoreMesh(core_axis_name="c", subcore_axis_name="s")
    return pl.kernel(
        _route_sc_body,
        out_type=[
            jax.ShapeDtypeStruct((N_TOKENS,), jnp.int32),
            jax.ShapeDtypeStruct((L,), jnp.int32),
        ],
        mesh=mesh,
        compiler_params=pltpu.CompilerParams(needs_layout_passes=False),
        scratch_types=[
            pltpu.VMEM((N_TOKENS,), jnp.int32),
            pltpu.VMEM((N_TOKENS,), jnp.float32),
            pltpu.VMEM((TPW,), jnp.int32),
            pltpu.VMEM((L,), jnp.int32),
        ],
    )(expert_idx, gate)


BT3 = 256
NBLK3 = N_TOKENS // BT3


def _expand_body(slot_ref, gate_ref, disp_ref, comb_ref):
    slot = slot_ref[...].reshape(BT3, 1, 1)
    gate = gate_ref[...].reshape(BT3, 1, 1)
    shp = (BT3, NUM_EXPERTS, CAP)
    ci = (lax.broadcasted_iota(jnp.int32, shp, 1) * CAP
          + lax.broadcasted_iota(jnp.int32, shp, 2))
    m = ci == slot
    disp_ref[...] = m.astype(jnp.float32)
    comb_ref[...] = jnp.where(m, gate, 0.0)


def _run_expand(slot2, gate2):
    return pl.pallas_call(
        _expand_body,
        grid=(NBLK3,),
        in_specs=[
            pl.BlockSpec((BT3, 1), lambda b: (b, 0)),
            pl.BlockSpec((BT3, 1), lambda b: (b, 0)),
        ],
        out_specs=[
            pl.BlockSpec((BT3, NUM_EXPERTS, CAP), lambda b: (b, 0, 0)),
            pl.BlockSpec((BT3, NUM_EXPERTS, CAP), lambda b: (b, 0, 0)),
        ],
        out_shape=[
            jax.ShapeDtypeStruct((N_TOKENS, NUM_EXPERTS, CAP), jnp.float32),
            jax.ShapeDtypeStruct((N_TOKENS, NUM_EXPERTS, CAP), jnp.float32),
        ],
    )(slot2, gate2)


def kernel(hidden_states, W):
    B, S, D = hidden_states.shape
    n = B * S
    E = W.shape[0]
    assert n == N_TOKENS and D == HIDDEN_DIM and E == NUM_EXPERTS

    hidden_flat = hidden_states.reshape(n, D)
    wt = W.T

    probs, gate2, idx2, partials = _run_router(hidden_flat, wt)

    slots, counts16 = _run_route_sc(idx2.reshape(n), gate2.reshape(n))
    counts = counts16[:E]

    dispatch, combine = _run_expand(slots.reshape(n, 1), gate2)

    usage = jnp.minimum(counts, CAP).astype(jnp.float32)
    psum = partials[:, 0, :E].sum(axis=0)
    zsum = partials[:, 0, E].sum()
    prob_per_expert = psum / n
    usage_per_expert = counts.astype(jnp.float32) / n
    load_loss = E * jnp.sum(prob_per_expert * usage_per_expert)
    z_loss = zsum / (n * E)
    dropped = jnp.maximum(0.0, n - usage.sum())

    return (dispatch, combine, jnp.array(CAP, dtype=jnp.int32), probs, usage,
            load_loss, z_loss, dropped)


# speedup vs baseline: 10.1539x; 1.0168x over previous
import functools

import jax
import jax.numpy as jnp
from jax import lax
from jax.experimental import pallas as pl
from jax.experimental.pallas import tpu as pltpu
from jax.experimental.pallas import tpu_sc as plsc

N_TOKENS = 4096
HIDDEN_DIM = 2048
NUM_EXPERTS = 8
CAPACITY_FACTOR = 1.25

NC = 2
NS = 16
L = 16
NW = NC * NS
TPW = N_TOKENS // NW
CPW = TPW // L
NCHUNKS = N_TOKENS // L

BT = 1024
NBLK = N_TOKENS // BT


def _capacity(num_tokens: int, num_experts: int) -> int:
    min_capacity_per_expert = 4
    base_capacity = int(CAPACITY_FACTOR * num_tokens / num_experts)
    if base_capacity < min_capacity_per_expert:
        adjusted_cf = min_capacity_per_expert * num_experts / max(num_tokens, 1)
        return max(min_capacity_per_expert,
                   int(adjusted_cf * num_tokens / num_experts))
    return base_capacity


CAP = _capacity(N_TOKENS, NUM_EXPERTS)
ECAP = NUM_EXPERTS * CAP
SENTINEL = ECAP

COLB = 2560
NCOL = ECAP // COLB


def _router_body(x_ref, wt_ref, probs_ref, gate_ref, idx_ref, part_ref):
    x = x_ref[...]
    wt = wt_ref[...]
    logits = jnp.dot(x, wt, preferred_element_type=jnp.float32)
    m = jnp.max(logits, axis=-1, keepdims=True)
    e = jnp.exp(logits - m)
    probs = e / jnp.sum(e, axis=-1, keepdims=True)
    probs_ref[...] = probs
    gate = jnp.max(probs, axis=-1, keepdims=True)
    gate_ref[...] = gate
    eiota = lax.broadcasted_iota(jnp.int32, probs.shape, 1)
    idx = jnp.min(jnp.where(probs == gate, eiota, NUM_EXPERTS),
                  axis=-1, keepdims=True)
    idx_ref[...] = idx
    colsum = jnp.sum(probs, axis=0)
    zsum = jnp.sum(logits * logits)
    row = jnp.concatenate([colsum, jnp.full((8,), zsum, jnp.float32)], axis=0)
    part_ref[...] = row.reshape(1, 1, 16)


def _run_router(hidden_flat, wt):
    return pl.pallas_call(
        _router_body,
        grid=(NBLK,),
        in_specs=[
            pl.BlockSpec((BT, HIDDEN_DIM), lambda b: (b, 0)),
            pl.BlockSpec((HIDDEN_DIM, NUM_EXPERTS), lambda b: (0, 0)),
        ],
        out_specs=[
            pl.BlockSpec((BT, NUM_EXPERTS), lambda b: (b, 0)),
            pl.BlockSpec((BT, 1), lambda b: (b, 0)),
            pl.BlockSpec((BT, 1), lambda b: (b, 0)),
            pl.BlockSpec((1, 1, 16), lambda b: (b, 0, 0)),
        ],
        out_shape=[
            jax.ShapeDtypeStruct((N_TOKENS, NUM_EXPERTS), jnp.float32),
            jax.ShapeDtypeStruct((N_TOKENS, 1), jnp.float32),
            jax.ShapeDtypeStruct((N_TOKENS, 1), jnp.int32),
            jax.ShapeDtypeStruct((NBLK, 1, 16), jnp.float32),
        ],
    )(hidden_flat, wt)


def _route_sc_body(idx_hbm, gate_hbm, slot_hbm, counts_hbm, e_v, g_v, out_v,
                   cnt_v):
    wid = lax.axis_index("s") * NC + lax.axis_index("c")
    base = wid * TPW
    my_first = wid * CPW

    pltpu.sync_copy(idx_hbm, e_v)
    pltpu.sync_copy(gate_hbm, g_v)

    zero = jnp.int32(0)
    zv = jnp.zeros((L,), jnp.int32)

    def count_body(c, carry):
        tot, pre = carry
        ev = e_v[pl.ds(c * L, L)]
        in_pre = jnp.where(c < my_first, jnp.int32(1), zero)
        new_tot = []
        new_pre = []
        for k in range(NUM_EXPERTS):
            pc = plsc.all_reduce_population_count(ev == k)
            new_tot.append(tot[k] + pc)
            new_pre.append(pre[k] + pc * in_pre)
        return tuple(new_tot), tuple(new_pre)

    init = (tuple(zv for _ in range(NUM_EXPERTS)),
            tuple(zv for _ in range(NUM_EXPERTS)))
    tot_v, pre_v = lax.fori_loop(0, NCHUNKS, count_body, init)
    tot = [tot_v[k][0] for k in range(NUM_EXPERTS)]
    pre = [pre_v[k][0] for k in range(NUM_EXPERTS)]

    ov = [tot[k] > CAP for k in range(NUM_EXPERTS)]
    any_ov = functools.reduce(jnp.logical_or, ov)

    lane = lax.broadcasted_iota(jnp.int32, (L,), 0)

    evs = [e_v[pl.ds((my_first + v) * L, L)] for v in range(CPW)]
    gvs = [g_v[pl.ds((my_first + v) * L, L)] for v in range(CPW)]

    run = list(pre)
    pos_vecs = []
    for v in range(CPW):
        ev = evs[v]
        pos = jnp.zeros((L,), jnp.int32)
        for k in range(NUM_EXPERTS):
            m = ev == k
            mi = m.astype(jnp.int32)
            cs = plsc.cumsum(mi)
            pos = jnp.where(m, run[k] + cs - 1, pos)
            run[k] = run[k] + plsc.all_reduce_population_count(m)[0]
        pos_vecs.append(pos)

    def grank_true(_):
        def pair_body(c, accs):
            eu_vec = e_v[pl.ds(c * L, L)]
            gu_vec = g_v[pl.ds(c * L, L)]
            accs = list(accs)
            for j in range(L):
                eu = eu_vec[j]
                gu = gu_vec[j]
                u = c * L + j
                for v in range(CPW):
                    tvec = base + v * L + lane
                    same = evs[v] == eu
                    beat = (gu > gvs[v]) | ((gu == gvs[v]) & (u < tvec))
                    accs[v] = accs[v] + (same & beat).astype(jnp.int32)
            return tuple(accs)

        return lax.fori_loop(
            0, NCHUNKS, pair_body,
            tuple(jnp.zeros((L,), jnp.int32) for _ in range(CPW)))

    def grank_false(_):
        return tuple(jnp.zeros((L,), jnp.int32) for _ in range(CPW))

    granks = lax.cond(any_ov, grank_true, grank_false, None)

    for v in range(CPW):
        ev = evs[v]
        ovv = jnp.zeros((L,), jnp.int32)
        for k in range(NUM_EXPERTS):
            ovv = jnp.where(ev == k, jnp.where(ov[k], jnp.int32(1), zero), ovv)
        ovb = ovv == 1
        slot = jnp.where(ovb, granks[v], pos_vecs[v])
        keep = jnp.where(ovb, granks[v] < CAP, True)
        flat = jnp.where(keep, ev * CAP + slot, jnp.int32(SENTINEL))
        out_v[pl.ds(v * L, L)] = flat

    pltpu.sync_copy(out_v, slot_hbm.at[pl.ds(base, TPW)])

    @pl.when(wid == 0)
    def _():
        cvec = jnp.zeros((L,), jnp.int32)
        for k in range(NUM_EXPERTS):
            cvec = jnp.where(lane == k, tot[k], cvec)
        cnt_v[...] = cvec
        pltpu.sync_copy(cnt_v, counts_hbm)


def _run_route_sc(expert_idx, gate):
    mesh = plsc.VectorSubcoreMesh(core_axis_name="c", subcore_axis_name="s")
    return pl.kernel(
        _route_sc_body,
        out_type=[
            jax.ShapeDtypeStruct((N_TOKENS,), jnp.int32),
            jax.ShapeDtypeStruct((L,), jnp.int32),
        ],
        mesh=mesh,
        compiler_params=pltpu.CompilerParams(needs_layout_passes=False),
        scratch_types=[
            pltpu.VMEM((N_TOKENS,), jnp.int32),
            pltpu.VMEM((N_TOKENS,), jnp.float32),
            pltpu.VMEM((TPW,), jnp.int32),
            pltpu.VMEM((L,), jnp.int32),
        ],
    )(expert_idx, gate)


BT3 = 256
NBLK3 = N_TOKENS // BT3


def _expand_body(slot_ref, gate_ref, disp_ref, comb_ref):
    slot = slot_ref[...].reshape(BT3, 1, 1)
    gate = gate_ref[...].reshape(BT3, 1, 1)
    shp = (BT3, NUM_EXPERTS, CAP)
    ci = (lax.broadcasted_iota(jnp.int32, shp, 1) * CAP
          + lax.broadcasted_iota(jnp.int32, shp, 2))
    m = ci == slot
    disp_ref[...] = m.astype(jnp.float32)
    comb_ref[...] = jnp.where(m, gate, 0.0)


def _run_expand(slot2, gate2):
    return pl.pallas_call(
        _expand_body,
        grid=(NBLK3,),
        in_specs=[
            pl.BlockSpec((BT3, 1), lambda b: (b, 0)),
            pl.BlockSpec((BT3, 1), lambda b: (b, 0)),
        ],
        out_specs=[
            pl.BlockSpec((BT3, NUM_EXPERTS, CAP), lambda b: (b, 0, 0)),
            pl.BlockSpec((BT3, NUM_EXPERTS, CAP), lambda b: (b, 0, 0)),
        ],
        out_shape=[
            jax.ShapeDtypeStruct((N_TOKENS, NUM_EXPERTS, CAP), jnp.float32),
            jax.ShapeDtypeStruct((N_TOKENS, NUM_EXPERTS, CAP), jnp.float32),
        ],
    )(slot2, gate2)


def kernel(hidden_states, W):
    B, S, D = hidden_states.shape
    n = B * S
    E = W.shape[0]
    assert n == N_TOKENS and D == HIDDEN_DIM and E == NUM_EXPERTS

    hidden_flat = hidden_states.reshape(n, D)
    wt = W.T

    probs, gate2, idx2, partials = _run_router(hidden_flat, wt)

    slots, counts16 = _run_route_sc(idx2.reshape(n), gate2.reshape(n))
    counts = counts16[:E]

    dispatch, combine = _run_expand(slots.reshape(n, 1), gate2)

    usage = jnp.minimum(counts, CAP).astype(jnp.float32)
    psum = partials[:, 0, :E].sum(axis=0)
    zsum = partials[:, 0, E].sum()
    prob_per_expert = psum / n
    usage_per_expert = counts.astype(jnp.float32) / n
    load_loss = E * jnp.sum(prob_per_expert * usage_per_expert)
    z_loss = zsum / (n * E)
    dropped = jnp.maximum(0.0, n - usage.sum())

    return (dispatch, combine, jnp.array(CAP, dtype=jnp.int32), probs, usage,
            load_loss, z_loss, dropped)
